# baseline (speedup 1.0000x reference)
import functools

import jax
import jax.numpy as jnp
from jax import lax
from jax.experimental import pallas as pl
from jax.experimental.pallas import tpu as pltpu
from jax.experimental.pallas import tpu_sc as plsc

NC = 2
NS = 16
NW = NC * NS
CH = 128

_MESH = plsc.VectorSubcoreMesh(
    core_axis_name="c", subcore_axis_name="s", num_cores=NC, num_subcores=NS
)


def _tile_rows(n):
    base = ((n + NS - 1) // NS + 7) // 8 * 8
    last = n - base * (NS - 1)
    assert 0 < last <= base and last % 8 == 0
    return base, last


def _plan(n, e):
    capw = -(-e // (NW * CH))
    ep = NW * capw * CH
    h0 = (n // 2 + 7) // 8 * 8
    na = h0 + 128
    capp = capw + 1
    n1 = (n + 127) // 128 * 128
    return capw, ep, h0, na, capp, n1


def _part_body(h0, esrc_hbm, edst_hbm, zeros1, psrc_hbm, pdst_hbm, cnt_hbm,
               degp_hbm, esrc_v, edst_v, ones_v, cnt_v,
               ps0, pd0, ps1, pd1, dsem, deg_sh):
    c = lax.axis_index("c")
    s = lax.axis_index("s")
    wid = c * NS + s
    capw = esrc_v.shape[0]
    n1 = deg_sh.shape[0]
    sr = ps0.shape[0]

    @pl.when(s == 0)
    def _():
        pltpu.sync_copy(zeros1, deg_sh)

    for k in range(CH // 16):
        ones_v[pl.ds(16 * k, 16)] = jnp.ones((16,), jnp.float32)
    pltpu.sync_copy(esrc_hbm.at[wid], esrc_v)
    pltpu.sync_copy(edst_hbm.at[wid], edst_v)
    plsc.subcore_barrier()

    def fire(j, carry):
        pltpu.async_copy(ones_v, deg_sh.at[edst_v.at[j]], dsem, add=True)
        return carry

    lax.fori_loop(0, capw, fire, 0)

    lane_id = lax.iota(jnp.int32, 16)

    def step(q, carry):
        cnt0, cnt1 = carry
        row = q // 8
        lane = (q % 8) * 16
        sv = esrc_v[row, pl.ds(lane, 16)]
        dv = edst_v[row, pl.ds(lane, 16)]
        packed = (sv << 14) | dv
        ks, vs = plsc.sort_key_val(dv, packed)
        m0 = ks < h0
        m1 = jnp.logical_not(m0)
        np0v = plsc.all_reduce_population_count(m0)
        svs = vs >> 14
        dvs = vs & 16383
        pos0 = cnt0 + lane_id
        pos1 = cnt1 + lane_id - np0v
        plsc.store_scatter(ps0, [pos0], svs, mask=m0)
        plsc.store_scatter(pd0, [pos0], dvs, mask=m0)
        plsc.store_scatter(ps1, [pos1], svs, mask=m1)
        plsc.store_scatter(pd1, [pos1], dvs - h0, mask=m1)
        np0 = np0v[0]
        return (cnt0 + np0, cnt1 + (16 - np0))

    cnt0, cnt1 = lax.fori_loop(0, capw * 8, step, (jnp.int32(0), jnp.int32(0)))

    trash_s = jnp.zeros((16,), jnp.int32)
    for k in range(CH // 16):
        trash_d = h0 + 16 * k + lane_id
        plsc.store_scatter(ps0, [cnt0 + 16 * k + lane_id], trash_s)
        plsc.store_scatter(pd0, [cnt0 + 16 * k + lane_id], trash_d)
        plsc.store_scatter(ps1, [cnt1 + 16 * k + lane_id], trash_s)
        plsc.store_scatter(pd1, [cnt1 + 16 * k + lane_id], trash_d)
    nch0 = (cnt0 + CH - 1) // CH
    nch1 = (cnt1 + CH - 1) // CH
    lane_id = lax.iota(jnp.int32, 16)
    cnt_v[...] = jnp.where(lane_id == 0, nch0,
                           jnp.where(lane_id == 1, nch1, 0))

    pltpu.sync_copy(ps0, psrc_hbm.at[pl.ds(pl.multiple_of(wid * sr, 128), sr)])
    pltpu.sync_copy(pd0, pdst_hbm.at[pl.ds(pl.multiple_of(wid * sr, 128), sr)])
    off1 = pl.multiple_of((NW + wid) * sr, 128)
    pltpu.sync_copy(ps1, psrc_hbm.at[pl.ds(off1, sr)])
    pltpu.sync_copy(pd1, pdst_hbm.at[pl.ds(off1, sr)])
    pltpu.sync_copy(cnt_v, cnt_hbm.at[pl.ds(pl.multiple_of(wid * 16, 8), 16)])

    def drain(j, carry):
        pltpu.make_async_copy(ones_v, deg_sh.at[edst_v.at[j]], dsem).wait()
        return carry

    lax.fori_loop(0, capw, drain, 0)
    plsc.subcore_barrier()

    @pl.when(s == 0)
    def _():
        pltpu.sync_copy(deg_sh,
                        degp_hbm.at[pl.ds(pl.multiple_of(c * n1, 128), n1)])


def _agg_body(h0, hs_hbm, psrc_hbm, pdst_hbm, cnt_hbm, zerosa, out_hbm,
              src_v, dst_v, cnt_v, rows_0, rows_1, rows_2, rows_3,
              gsem, ssem, acc_sh):
    c = lax.axis_index("c")
    s = lax.axis_index("s")
    wid = c * NS + s
    na = acc_sh.shape[0]
    n = out_hbm.shape[0]
    h1 = n - h0
    base, last = _tile_rows(na)
    off = pl.multiple_of(s * base, 8)
    lo_last = base * (NS - 1)
    rows = (rows_0, rows_1, rows_2, rows_3)

    @pl.when(s < NS - 1)
    def _():
        pltpu.sync_copy(zerosa.at[pl.ds(off, base)], acc_sh.at[pl.ds(off, base)])

    @pl.when(s == NS - 1)
    def _():
        pltpu.sync_copy(zerosa.at[pl.ds(lo_last, last)],
                        acc_sh.at[pl.ds(lo_last, last)])

    barriered = False
    for roff in (0, NS):
        reg = s + roff
        pltpu.sync_copy(psrc_hbm.at[c, reg], src_v)
        pltpu.sync_copy(pdst_hbm.at[c, reg], dst_v)
        pltpu.sync_copy(
            cnt_hbm.at[pl.ds(pl.multiple_of(reg * 16, 8), 16)], cnt_v)
        ncv = cnt_v[...]
        nch = jnp.where(c == 0, ncv[0], ncv[1])

        @pl.when(nch > 0)
        def _():
            pltpu.async_copy(hs_hbm.at[src_v.at[0]], rows[0], gsem.at[0])

        @pl.when(nch > 1)
        def _():
            pltpu.async_copy(hs_hbm.at[src_v.at[1]], rows[1], gsem.at[1])

        if not barriered:
            plsc.subcore_barrier()
            barriered = True

        def quad(g, carry, nch=nch, src_v=src_v, dst_v=dst_v):
            for u in range(4):
                t = 4 * g + u
                b_cur = u
                b_pre = (u + 2) % 4

                @pl.when((t >= 2) & (t < nch + 2))
                def _():
                    pltpu.make_async_copy(
                        rows[b_pre], acc_sh.at[dst_v.at[t - 2]],
                        ssem.at[b_pre]).wait()

                @pl.when(t + 2 < nch)
                def _():
                    pltpu.async_copy(hs_hbm.at[src_v.at[t + 2]], rows[b_pre],
                                     gsem.at[b_pre])

                @pl.when(t < nch)
                def _():
                    pltpu.make_async_copy(hs_hbm.at[src_v.at[t]], rows[b_cur],
                                          gsem.at[b_cur]).wait()
                    pltpu.async_copy(rows[b_cur], acc_sh.at[dst_v.at[t]],
                                     ssem.at[b_cur], add=True)

            return carry

        lax.fori_loop(0, (nch + 2 + 3) // 4, quad, 0)

    plsc.subcore_barrier()

    @pl.when(s < NS - 1)
    def _():
        pltpu.sync_copy(acc_sh.at[pl.ds(off, base)],
                        out_hbm.at[pl.ds(pl.multiple_of(c * h0 + s * base, 8),
                                         base)])

    @pl.when((s == NS - 1) & (c == 0))
    def _():
        pltpu.sync_copy(acc_sh.at[pl.ds(lo_last, h0 - lo_last)],
                        out_hbm.at[pl.ds(lo_last, h0 - lo_last)])

    @pl.when((s == NS - 1) & (c == 1))
    def _():
        pltpu.sync_copy(acc_sh.at[pl.ds(lo_last, h1 - lo_last)],
                        out_hbm.at[pl.ds(h0 + lo_last, h1 - lo_last)])


def _make_sc_kernels(n, d, e):
    capw, ep, h0, na, capp, n1 = _plan(n, e)
    sr = capp * CH
    part_k = pl.kernel(
        functools.partial(_part_body, h0),
        compiler_params=pltpu.CompilerParams(needs_layout_passes=False),
        out_type=(
            jax.ShapeDtypeStruct((NC * NW * sr,), jnp.int32),
            jax.ShapeDtypeStruct((NC * NW * sr,), jnp.int32),
            jax.ShapeDtypeStruct((NW * 16,), jnp.int32),
            jax.ShapeDtypeStruct((NC * n1,), jnp.float32),
        ),
        mesh=_MESH,
        scratch_types=[
            pltpu.VMEM((capw, CH), jnp.int32),
            pltpu.VMEM((capw, CH), jnp.int32),
            pltpu.VMEM((CH,), jnp.float32),
            pltpu.VMEM((16,), jnp.int32),
            pltpu.VMEM((sr,), jnp.int32),
            pltpu.VMEM((sr,), jnp.int32),
            pltpu.VMEM((sr,), jnp.int32),
            pltpu.VMEM((sr,), jnp.int32),
            pltpu.SemaphoreType.DMA,
            pltpu.VMEM_SHARED((n1,), jnp.float32),
        ],
    )
    agg_k = pl.kernel(
        functools.partial(_agg_body, h0),
        compiler_params=pltpu.CompilerParams(needs_layout_passes=False,
                                             use_tc_tiling_on_sc=False),
        out_type=jax.ShapeDtypeStruct((n, d), jnp.float32),
        mesh=_MESH,
        scratch_types=[
            pltpu.VMEM((capp, CH), jnp.int32),
            pltpu.VMEM((capp, CH), jnp.int32),
            pltpu.VMEM((16,), jnp.int32),
            pltpu.VMEM((CH, d), jnp.float32),
            pltpu.VMEM((CH, d), jnp.float32),
            pltpu.VMEM((CH, d), jnp.float32),
            pltpu.VMEM((CH, d), jnp.float32),
            pltpu.SemaphoreType.DMA((4,)),
            pltpu.SemaphoreType.DMA((4,)),
            pltpu.VMEM_SHARED((na, d), jnp.float32),
        ],
    )
    return part_k, agg_k


def _dis_col(degp_ref, nrows):
    deg = degp_ref[0] + degp_ref[1] + 1.0
    dis = lax.rsqrt(deg)
    return jnp.transpose(dis)[:nrows, :]


def _tc1_body(x_ref, w1_ref, degp_ref, h1s_ref):
    dis = _dis_col(degp_ref, x_ref.shape[0])
    h = jnp.dot(x_ref[...], w1_ref[...], preferred_element_type=jnp.float32)
    h1s_ref[...] = h * dis


def _tc2_body(acc_ref, h1s_ref, degp_ref, b1_ref, w2_ref, h2s_ref):
    dis = _dis_col(degp_ref, acc_ref.shape[0])
    pre = (acc_ref[...] + h1s_ref[...]) * dis + b1_ref[...]
    h1 = jnp.maximum(pre, 0.0)
    h2 = jnp.dot(h1, w2_ref[...], preferred_element_type=jnp.float32)
    h2s_ref[...] = h2 * dis


def _tc3_body(acc_ref, h2s_ref, degp_ref, b2_ref, out_ref):
    dis = _dis_col(degp_ref, acc_ref.shape[0])
    out_ref[...] = (acc_ref[...] + h2s_ref[...]) * dis + b2_ref[...]


def kernel(x, edge_index, W1, b1, W2, b2):
    n, _ = x.shape
    d_hid = W1.shape[1]
    d_out = W2.shape[1]
    e = edge_index.shape[1]
    capw, ep, h0, na, capp, n1 = _plan(n, e)
    sr = capp * CH

    ei = edge_index.astype(jnp.int32)
    pad = ep - e
    esrc = jnp.concatenate([ei[0], jnp.zeros((pad,), jnp.int32)])
    edst = jnp.concatenate(
        [ei[1], n + (jnp.arange(pad, dtype=jnp.int32) % 128)])
    esrc = esrc.reshape(NW, capw, CH)
    edst = edst.reshape(NW, capw, CH)
    zeros1 = jnp.zeros((n1,), jnp.float32)
    zerosa = jnp.zeros((na, d_hid), jnp.float32)

    part_k, agg_k = _make_sc_kernels(n, d_hid, e)

    psrc, pdst, cnt, degp = part_k(esrc, edst, zeros1)
    psrc = psrc.reshape(NC, NW, capp, CH)
    pdst = pdst.reshape(NC, NW, capp, CH)
    degp2 = degp.reshape(NC, 1, n1)

    tc1 = pl.pallas_call(
        _tc1_body,
        out_shape=jax.ShapeDtypeStruct((n, d_hid), jnp.float32),
    )
    h1s = tc1(x, W1, degp2)

    acc1 = agg_k(h1s, psrc, pdst, cnt, zerosa)

    tc2 = pl.pallas_call(
        _tc2_body,
        out_shape=jax.ShapeDtypeStruct((n, d_hid), jnp.float32),
    )
    h2s = tc2(acc1, h1s, degp2, b1.reshape(1, d_hid), W2)

    acc2 = agg_k(h2s, psrc, pdst, cnt, zerosa)

    tc3 = pl.pallas_call(
        _tc3_body,
        out_shape=jax.ShapeDtypeStruct((n, d_out), jnp.float32),
    )
    out = tc3(acc2, h2s, degp2, b2.reshape(1, d_out))
    return out


# speedup vs baseline: 2.6437x; 2.6437x over previous
import jax
import jax.numpy as jnp
from jax import lax
from jax.experimental import pallas as pl
from jax.experimental.pallas import tpu as pltpu
from jax.experimental.pallas import tpu_sc as plsc

NC = 2
NS = 16
NW = NC * NS
DH = 64

_MESH = plsc.VectorSubcoreMesh(
    core_axis_name="c", subcore_axis_name="s", num_cores=NC, num_subcores=NS
)


def _tile_rows(n):
    base = ((n + NS - 1) // NS + 7) // 8 * 8
    last = n - base * (NS - 1)
    assert 0 < last <= base and last % 8 == 0
    return base, last


def _deg_body(dst_hbm, zeros1, out_hbm, dst_v, ones_v, deg_sh, sem):
    c = lax.axis_index("c")
    s = lax.axis_index("s")
    wid = c * NS + s
    nch = dst_v.shape[0]
    n1 = deg_sh.shape[0]

    @pl.when(s == 0)
    def _():
        pltpu.sync_copy(zeros1, deg_sh)

    for k in range(ones_v.shape[0] // 16):
        ones_v[pl.ds(16 * k, 16)] = jnp.ones((16,), jnp.float32)
    pltpu.sync_copy(dst_hbm.at[wid], dst_v)
    plsc.subcore_barrier()

    def step(j, carry):
        pltpu.sync_copy(ones_v, deg_sh.at[dst_v.at[j]], add=True)
        return carry

    lax.fori_loop(0, nch, step, 0)
    plsc.subcore_barrier()

    @pl.when(s == 0)
    def _():
        pltpu.sync_copy(deg_sh, out_hbm.at[pl.ds(pl.multiple_of(c * n1, 128), n1)])


def _agg_body(hs0, hs1, src_hbm, dst_hbm, zeros2, out_hbm,
              src_v, dst_v, rows_0, rows_1, rows_2, rows_3,
              gsem, ssem, acc_sh):
    c = lax.axis_index("c")
    s = lax.axis_index("s")
    wid = c * NS + s
    nch = src_v.shape[0]
    assert nch >= 2
    n = acc_sh.shape[0]
    base, last = _tile_rows(n)
    off = pl.multiple_of(s * base, 8)
    lo_last = base * (NS - 1)
    rows = (rows_0, rows_1, rows_2, rows_3)

    pltpu.sync_copy(src_hbm.at[wid], src_v)
    pltpu.sync_copy(dst_hbm.at[wid], dst_v)

    for half, hs in enumerate((hs0, hs1)):
        @pl.when(s < NS - 1)
        def _():
            pltpu.sync_copy(zeros2.at[pl.ds(off, base)],
                            acc_sh.at[pl.ds(off, base)])

        @pl.when(s == NS - 1)
        def _():
            pltpu.sync_copy(zeros2.at[pl.ds(lo_last, last)],
                            acc_sh.at[pl.ds(lo_last, last)])

        pltpu.async_copy(hs.at[src_v.at[0]], rows[0], gsem.at[0])
        pltpu.async_copy(hs.at[src_v.at[1]], rows[1], gsem.at[1])
        plsc.subcore_barrier()

        def quad(g, carry, hs=hs):
            for u in range(4):
                t = 4 * g + u
                b_cur = u
                b_pre = (u + 2) % 4

                @pl.when((t >= 2) & (t < nch + 2))
                def _():
                    pltpu.make_async_copy(
                        rows[b_pre], acc_sh.at[dst_v.at[t - 2]],
                        ssem.at[b_pre]).wait()

                @pl.when(t + 2 < nch)
                def _():
                    pltpu.async_copy(hs.at[src_v.at[t + 2]], rows[b_pre],
                                     gsem.at[b_pre])

                @pl.when(t < nch)
                def _():
                    pltpu.make_async_copy(hs.at[src_v.at[t]], rows[b_cur],
                                          gsem.at[b_cur]).wait()
                    pltpu.async_copy(rows[b_cur], acc_sh.at[dst_v.at[t]],
                                     ssem.at[b_cur], add=True)

            return carry

        lax.fori_loop(0, (nch + 2 + 3) // 4, quad, 0)
        plsc.subcore_barrier()

        @pl.when(s < NS - 1)
        def _():
            pltpu.sync_copy(acc_sh.at[pl.ds(off, base)],
                            out_hbm.at[half, c, pl.ds(off, base)])

        @pl.when(s == NS - 1)
        def _():
            pltpu.sync_copy(acc_sh.at[pl.ds(lo_last, last)],
                            out_hbm.at[half, c, pl.ds(lo_last, last)])


def _make_sc_kernels(n, nch, ch):
    n1 = (n + 127) // 128 * 128
    deg_k = pl.kernel(
        _deg_body,
        out_type=jax.ShapeDtypeStruct((NC * n1,), jnp.float32),
        mesh=_MESH,
        scratch_types=[
            pltpu.VMEM((nch, ch), jnp.int32),
            pltpu.VMEM((ch,), jnp.float32),
            pltpu.VMEM_SHARED((n1,), jnp.float32),
            pltpu.SemaphoreType.DMA,
        ],
    )
    agg_k = pl.kernel(
        _agg_body,
        out_type=jax.ShapeDtypeStruct((2, NC, n, DH), jnp.float32),
        mesh=_MESH,
        compiler_params=pltpu.CompilerParams(use_tc_tiling_on_sc=False),
        scratch_types=[
            pltpu.VMEM((nch, ch), jnp.int32),
            pltpu.VMEM((nch, ch), jnp.int32),
            pltpu.VMEM((ch, DH), jnp.float32),
            pltpu.VMEM((ch, DH), jnp.float32),
            pltpu.VMEM((ch, DH), jnp.float32),
            pltpu.VMEM((ch, DH), jnp.float32),
            pltpu.SemaphoreType.DMA((4,)),
            pltpu.SemaphoreType.DMA((4,)),
            pltpu.VMEM_SHARED((n, DH), jnp.float32),
        ],
    )
    return deg_k, agg_k


def _dis_col(degp_ref, nrows):
    deg = degp_ref[0] + degp_ref[1] + 1.0
    dis = lax.rsqrt(deg)
    return jnp.transpose(dis)[:nrows, :]


def _tc1_body(x_ref, w1_ref, degp_ref, h1s0_ref, h1s1_ref):
    dis = _dis_col(degp_ref, x_ref.shape[0])
    h = jnp.dot(x_ref[...], w1_ref[...], preferred_element_type=jnp.float32)
    hs = h * dis
    h1s0_ref[...] = hs[:, :DH]
    h1s1_ref[...] = hs[:, DH:]


def _tc2_body(agg_ref, h1s0_ref, h1s1_ref, degp_ref, b1_ref, w2_ref,
              h2s0_ref, h2s1_ref):
    dis = _dis_col(degp_ref, agg_ref.shape[2])
    b1 = b1_ref[...]
    pre0 = (agg_ref[0, 0] + agg_ref[0, 1] + h1s0_ref[...]) * dis + b1[:, :DH]
    pre1 = (agg_ref[1, 0] + agg_ref[1, 1] + h1s1_ref[...]) * dis + b1[:, DH:]
    h1 = jnp.concatenate([jnp.maximum(pre0, 0.0), jnp.maximum(pre1, 0.0)],
                         axis=1)
    h2 = jnp.dot(h1, w2_ref[...], preferred_element_type=jnp.float32)
    hs = h2 * dis
    h2s0_ref[...] = hs[:, :DH]
    h2s1_ref[...] = hs[:, DH:]


def _tc3_body(agg_ref, h2s0_ref, h2s1_ref, degp_ref, b2_ref, out_ref):
    dis = _dis_col(degp_ref, agg_ref.shape[2])
    b2 = b2_ref[...]
    o0 = (agg_ref[0, 0] + agg_ref[0, 1] + h2s0_ref[...]) * dis + b2[:, :DH]
    o1 = (agg_ref[1, 0] + agg_ref[1, 1] + h2s1_ref[...]) * dis + b2[:, DH:]
    out_ref[...] = jnp.concatenate([o0, o1], axis=1)


def kernel(x, edge_index, W1, b1, W2, b2):
    n, _ = x.shape
    d_hid = W1.shape[1]
    d_out = W2.shape[1]
    e = edge_index.shape[1]
    assert d_hid == 2 * DH and d_out == 2 * DH

    per_w = e // NW
    ch = 80
    while per_w % ch:
        ch -= 8
    nch = per_w // ch

    ei = edge_index.astype(jnp.int32)
    src = ei[0].reshape(NW, nch, ch)
    dst = ei[1].reshape(NW, nch, ch)
    n1 = (n + 127) // 128 * 128
    zeros1 = jnp.zeros((n1,), jnp.float32)
    zeros2 = jnp.zeros((n, DH), jnp.float32)

    deg_k, agg_k = _make_sc_kernels(n, nch, ch)

    degp = deg_k(dst, zeros1)
    degp2 = degp.reshape(NC, 1, n1)

    tc1 = pl.pallas_call(
        _tc1_body,
        out_shape=(
            jax.ShapeDtypeStruct((n, DH), jnp.float32),
            jax.ShapeDtypeStruct((n, DH), jnp.float32),
        ),
    )
    h1s0, h1s1 = tc1(x, W1, degp2)

    agg1 = agg_k(h1s0, h1s1, src, dst, zeros2)

    tc2 = pl.pallas_call(
        _tc2_body,
        out_shape=(
            jax.ShapeDtypeStruct((n, DH), jnp.float32),
            jax.ShapeDtypeStruct((n, DH), jnp.float32),
        ),
    )
    h2s0, h2s1 = tc2(agg1, h1s0, h1s1, degp2, b1.reshape(1, d_hid), W2)

    agg2 = agg_k(h2s0, h2s1, src, dst, zeros2)

    tc3 = pl.pallas_call(
        _tc3_body,
        out_shape=jax.ShapeDtypeStruct((n, d_out), jnp.float32),
    )
    out = tc3(agg2, h2s0, h2s1, degp2, b2.reshape(1, d_out))
    return out


# speedup vs baseline: 2.6963x; 1.0199x over previous
import jax
import jax.numpy as jnp
from jax import lax
from jax.experimental import pallas as pl
from jax.experimental.pallas import tpu as pltpu
from jax.experimental.pallas import tpu_sc as plsc

NC = 2
NS = 16
NW = NC * NS
DH = 64

_MESH = plsc.VectorSubcoreMesh(
    core_axis_name="c", subcore_axis_name="s", num_cores=NC, num_subcores=NS
)


def _tile_rows(n):
    base = ((n + NS - 1) // NS + 7) // 8 * 8
    last = n - base * (NS - 1)
    assert 0 < last <= base and last % 8 == 0
    return base, last


def _deg_body(dst_hbm, zeros1, out_hbm, dst_v, ones_v, deg_sh, sem):
    c = lax.axis_index("c")
    s = lax.axis_index("s")
    wid = c * NS + s
    nch = dst_v.shape[0]
    n1 = deg_sh.shape[0]

    @pl.when(s == 0)
    def _():
        pltpu.sync_copy(zeros1, deg_sh)

    for k in range(ones_v.shape[0] // 16):
        ones_v[pl.ds(16 * k, 16)] = jnp.ones((16,), jnp.float32)
    pltpu.sync_copy(dst_hbm.at[wid], dst_v)
    plsc.subcore_barrier()

    def fire(j, carry):
        pltpu.async_copy(ones_v, deg_sh.at[dst_v.at[j]], sem, add=True)
        return carry

    lax.fori_loop(0, nch, fire, 0)

    def drain(j, carry):
        pltpu.make_async_copy(ones_v, deg_sh.at[dst_v.at[j]], sem).wait()
        return carry

    lax.fori_loop(0, nch, drain, 0)
    plsc.subcore_barrier()

    @pl.when(s == 0)
    def _():
        pltpu.sync_copy(deg_sh, out_hbm.at[pl.ds(pl.multiple_of(c * n1, 128), n1)])


def _agg_body(hs0, hs1, src_hbm, dst_hbm, zeros2, out_hbm,
              src_v, dst_v, rows_0, rows_1, rows_2, rows_3,
              gsem, ssem, acc_sh):
    c = lax.axis_index("c")
    s = lax.axis_index("s")
    wid = c * NS + s
    nch = src_v.shape[0]
    assert nch >= 2
    n = acc_sh.shape[0]
    base, last = _tile_rows(n)
    off = pl.multiple_of(s * base, 8)
    lo_last = base * (NS - 1)
    rows = (rows_0, rows_1, rows_2, rows_3)

    pltpu.sync_copy(src_hbm.at[wid], src_v)
    pltpu.sync_copy(dst_hbm.at[wid], dst_v)

    for half, hs in enumerate((hs0, hs1)):
        @pl.when(s < NS - 1)
        def _():
            pltpu.sync_copy(zeros2.at[pl.ds(off, base)],
                            acc_sh.at[pl.ds(off, base)])

        @pl.when(s == NS - 1)
        def _():
            pltpu.sync_copy(zeros2.at[pl.ds(lo_last, last)],
                            acc_sh.at[pl.ds(lo_last, last)])

        pltpu.async_copy(hs.at[src_v.at[0]], rows[0], gsem.at[0])
        pltpu.async_copy(hs.at[src_v.at[1]], rows[1], gsem.at[1])
        plsc.subcore_barrier()

        def quad(g, carry, hs=hs):
            for u in range(4):
                t = 4 * g + u
                b_cur = u
                b_pre = (u + 2) % 4

                @pl.when((t >= 2) & (t < nch + 2))
                def _():
                    pltpu.make_async_copy(
                        rows[b_pre], acc_sh.at[dst_v.at[t - 2]],
                        ssem.at[b_pre]).wait()

                @pl.when(t + 2 < nch)
                def _():
                    pltpu.async_copy(hs.at[src_v.at[t + 2]], rows[b_pre],
                                     gsem.at[b_pre])

                @pl.when(t < nch)
                def _():
                    pltpu.make_async_copy(hs.at[src_v.at[t]], rows[b_cur],
                                          gsem.at[b_cur]).wait()
                    pltpu.async_copy(rows[b_cur], acc_sh.at[dst_v.at[t]],
                                     ssem.at[b_cur], add=True)

            return carry

        lax.fori_loop(0, (nch + 2 + 3) // 4, quad, 0)
        plsc.subcore_barrier()

        @pl.when(s < NS - 1)
        def _():
            pltpu.sync_copy(acc_sh.at[pl.ds(off, base)],
                            out_hbm.at[half, c, pl.ds(off, base)])

        @pl.when(s == NS - 1)
        def _():
            pltpu.sync_copy(acc_sh.at[pl.ds(lo_last, last)],
                            out_hbm.at[half, c, pl.ds(lo_last, last)])


def _make_sc_kernels(n, nch, ch):
    n1 = (n + 127) // 128 * 128
    deg_k = pl.kernel(
        _deg_body,
        out_type=jax.ShapeDtypeStruct((NC * n1,), jnp.float32),
        mesh=_MESH,
        scratch_types=[
            pltpu.VMEM((nch, ch), jnp.int32),
            pltpu.VMEM((ch,), jnp.float32),
            pltpu.VMEM_SHARED((n1,), jnp.float32),
            pltpu.SemaphoreType.DMA,
        ],
    )
    agg_k = pl.kernel(
        _agg_body,
        out_type=jax.ShapeDtypeStruct((2, NC, n, DH), jnp.float32),
        mesh=_MESH,
        compiler_params=pltpu.CompilerParams(use_tc_tiling_on_sc=False),
        scratch_types=[
            pltpu.VMEM((nch, ch), jnp.int32),
            pltpu.VMEM((nch, ch), jnp.int32),
            pltpu.VMEM((ch, DH), jnp.float32),
            pltpu.VMEM((ch, DH), jnp.float32),
            pltpu.VMEM((ch, DH), jnp.float32),
            pltpu.VMEM((ch, DH), jnp.float32),
            pltpu.SemaphoreType.DMA((4,)),
            pltpu.SemaphoreType.DMA((4,)),
            pltpu.VMEM_SHARED((n, DH), jnp.float32),
        ],
    )
    return deg_k, agg_k


def _dis_col(degp_ref, nrows):
    deg = degp_ref[0] + degp_ref[1] + 1.0
    dis = lax.rsqrt(deg)
    return jnp.transpose(dis)[:nrows, :]


def _tc1_body(x_ref, w1_ref, degp_ref, h1s0_ref, h1s1_ref):
    dis = _dis_col(degp_ref, x_ref.shape[0])
    h = jnp.dot(x_ref[...], w1_ref[...], preferred_element_type=jnp.float32)
    hs = h * dis
    h1s0_ref[...] = hs[:, :DH]
    h1s1_ref[...] = hs[:, DH:]


def _tc2_body(agg_ref, h1s0_ref, h1s1_ref, degp_ref, b1_ref, w2_ref,
              h2s0_ref, h2s1_ref):
    dis = _dis_col(degp_ref, agg_ref.shape[2])
    b1 = b1_ref[...]
    pre0 = (agg_ref[0, 0] + agg_ref[0, 1] + h1s0_ref[...]) * dis + b1[:, :DH]
    pre1 = (agg_ref[1, 0] + agg_ref[1, 1] + h1s1_ref[...]) * dis + b1[:, DH:]
    h1 = jnp.concatenate([jnp.maximum(pre0, 0.0), jnp.maximum(pre1, 0.0)],
                         axis=1)
    h2 = jnp.dot(h1, w2_ref[...], preferred_element_type=jnp.float32)
    hs = h2 * dis
    h2s0_ref[...] = hs[:, :DH]
    h2s1_ref[...] = hs[:, DH:]


def _tc3_body(agg_ref, h2s0_ref, h2s1_ref, degp_ref, b2_ref, out_ref):
    dis = _dis_col(degp_ref, agg_ref.shape[2])
    b2 = b2_ref[...]
    o0 = (agg_ref[0, 0] + agg_ref[0, 1] + h2s0_ref[...]) * dis + b2[:, :DH]
    o1 = (agg_ref[1, 0] + agg_ref[1, 1] + h2s1_ref[...]) * dis + b2[:, DH:]
    out_ref[...] = jnp.concatenate([o0, o1], axis=1)


def kernel(x, edge_index, W1, b1, W2, b2):
    n, _ = x.shape
    d_hid = W1.shape[1]
    d_out = W2.shape[1]
    e = edge_index.shape[1]
    assert d_hid == 2 * DH and d_out == 2 * DH

    per_w = e // NW
    ch = 80
    while per_w % ch:
        ch -= 8
    nch = per_w // ch

    ei = edge_index.astype(jnp.int32)
    src = ei[0].reshape(NW, nch, ch)
    dst = ei[1].reshape(NW, nch, ch)
    n1 = (n + 127) // 128 * 128
    zeros1 = jnp.zeros((n1,), jnp.float32)
    zeros2 = jnp.zeros((n, DH), jnp.float32)

    deg_k, agg_k = _make_sc_kernels(n, nch, ch)

    degp = deg_k(dst, zeros1)
    degp2 = degp.reshape(NC, 1, n1)

    tc1 = pl.pallas_call(
        _tc1_body,
        out_shape=(
            jax.ShapeDtypeStruct((n, DH), jnp.float32),
            jax.ShapeDtypeStruct((n, DH), jnp.float32),
        ),
    )
    h1s0, h1s1 = tc1(x, W1, degp2)

    agg1 = agg_k(h1s0, h1s1, src, dst, zeros2)

    tc2 = pl.pallas_call(
        _tc2_body,
        out_shape=(
            jax.ShapeDtypeStruct((n, DH), jnp.float32),
            jax.ShapeDtypeStruct((n, DH), jnp.float32),
        ),
    )
    h2s0, h2s1 = tc2(agg1, h1s0, h1s1, degp2, b1.reshape(1, d_hid), W2)

    agg2 = agg_k(h2s0, h2s1, src, dst, zeros2)

    tc3 = pl.pallas_call(
        _tc3_body,
        out_shape=jax.ShapeDtypeStruct((n, d_out), jnp.float32),
    )
    out = tc3(agg2, h2s0, h2s1, degp2, b2.reshape(1, d_out))
    return out


# speedup vs baseline: 3.2636x; 1.2104x over previous
import jax
import jax.numpy as jnp
from jax import lax
from jax.experimental import pallas as pl
from jax.experimental.pallas import tpu as pltpu
from jax.experimental.pallas import tpu_sc as plsc

NC = 2
NS = 16
NW = NC * NS
DH = 64

_MESH = plsc.VectorSubcoreMesh(
    core_axis_name="c", subcore_axis_name="s", num_cores=NC, num_subcores=NS
)


def _tile_rows(n):
    base = ((n + NS - 1) // NS + 7) // 8 * 8
    last = n - base * (NS - 1)
    assert 0 < last <= base and last % 8 == 0
    return base, last


def _deg_body(dst_hbm, zeros1, out_hbm, dst_v, ones_v, deg_sh, sem):
    c = lax.axis_index("c")
    s = lax.axis_index("s")
    wid = c * NS + s
    nch = dst_v.shape[0]
    n1 = deg_sh.shape[0]

    @pl.when(s == 0)
    def _():
        pltpu.sync_copy(zeros1, deg_sh)

    for k in range(ones_v.shape[0] // 16):
        ones_v[pl.ds(16 * k, 16)] = jnp.ones((16,), jnp.float32)
    pltpu.sync_copy(dst_hbm.at[wid], dst_v)
    plsc.subcore_barrier()

    def fire(j, carry):
        pltpu.async_copy(ones_v, deg_sh.at[dst_v.at[j]], sem, add=True)
        return carry

    lax.fori_loop(0, nch, fire, 0)

    def drain(j, carry):
        pltpu.make_async_copy(ones_v, deg_sh.at[dst_v.at[j]], sem).wait()
        return carry

    lax.fori_loop(0, nch, drain, 0)
    plsc.subcore_barrier()

    @pl.when(s == 0)
    def _():
        pltpu.sync_copy(deg_sh, out_hbm.at[pl.ds(pl.multiple_of(c * n1, 128), n1)])


def _agg_body(hs2_hbm, src_hbm, dst_hbm, zeros2, out_hbm,
              src_v, dst_v, s2a_v, s2b_v, rows_0, rows_1, rows_2, rows_3,
              gsem, ssem, acc_sh):
    c = lax.axis_index("c")
    s = lax.axis_index("s")
    wid = c * NS + s
    nch = src_v.shape[0]
    ch = src_v.shape[1]
    assert nch >= 2
    n = acc_sh.shape[0]
    base, last = _tile_rows(n)
    off = pl.multiple_of(s * base, 8)
    lo_last = base * (NS - 1)
    rows = (rows_0, rows_1, rows_2, rows_3)

    pltpu.sync_copy(src_hbm.at[wid], src_v)
    pltpu.sync_copy(dst_hbm.at[wid], dst_v)

    def xform(q, carry):
        row = q // (ch // 16)
        lane = (q % (ch // 16)) * 16
        v = src_v[row, pl.ds(lane, 16)]
        s2a_v[row, pl.ds(lane, 16)] = v * 2
        s2b_v[row, pl.ds(lane, 16)] = v * 2 + 1
        return carry

    lax.fori_loop(0, nch * (ch // 16), xform, 0)

    for half, s2_v in enumerate((s2a_v, s2b_v)):
        @pl.when(s < NS - 1)
        def _():
            pltpu.sync_copy(zeros2.at[pl.ds(off, base)],
                            acc_sh.at[pl.ds(off, base)])

        @pl.when(s == NS - 1)
        def _():
            pltpu.sync_copy(zeros2.at[pl.ds(lo_last, last)],
                            acc_sh.at[pl.ds(lo_last, last)])

        pltpu.async_copy(hs2_hbm.at[s2_v.at[0]], rows[0], gsem.at[0])
        pltpu.async_copy(hs2_hbm.at[s2_v.at[1]], rows[1], gsem.at[1])
        plsc.subcore_barrier()

        def quad(g, carry, s2_v=s2_v):
            for u in range(4):
                t = 4 * g + u
                b_cur = u
                b_pre = (u + 2) % 4

                @pl.when((t >= 2) & (t < nch + 2))
                def _():
                    pltpu.make_async_copy(
                        rows[b_pre], acc_sh.at[dst_v.at[t - 2]],
                        ssem.at[b_pre]).wait()

                @pl.when(t + 2 < nch)
                def _():
                    pltpu.async_copy(hs2_hbm.at[s2_v.at[t + 2]], rows[b_pre],
                                     gsem.at[b_pre])

                @pl.when(t < nch)
                def _():
                    pltpu.make_async_copy(hs2_hbm.at[s2_v.at[t]], rows[b_cur],
                                          gsem.at[b_cur]).wait()
                    pltpu.async_copy(rows[b_cur], acc_sh.at[dst_v.at[t]],
                                     ssem.at[b_cur], add=True)

            return carry

        lax.fori_loop(0, (nch + 2 + 3) // 4, quad, 0)
        plsc.subcore_barrier()

        @pl.when(s < NS - 1)
        def _():
            pltpu.sync_copy(acc_sh.at[pl.ds(off, base)],
                            out_hbm.at[c, pl.ds(off, base),
                                       pl.ds(half * DH, DH)])

        @pl.when(s == NS - 1)
        def _():
            pltpu.sync_copy(acc_sh.at[pl.ds(lo_last, last)],
                            out_hbm.at[c, pl.ds(lo_last, last),
                                       pl.ds(half * DH, DH)])


def _make_sc_kernels(n, nch, ch):
    n1 = (n + 127) // 128 * 128
    deg_k = pl.kernel(
        _deg_body,
        out_type=jax.ShapeDtypeStruct((NC * n1,), jnp.float32),
        mesh=_MESH,
        scratch_types=[
            pltpu.VMEM((nch, ch), jnp.int32),
            pltpu.VMEM((ch,), jnp.float32),
            pltpu.VMEM_SHARED((n1,), jnp.float32),
            pltpu.SemaphoreType.DMA,
        ],
    )
    agg_k = pl.kernel(
        _agg_body,
        out_type=jax.ShapeDtypeStruct((NC, n, 2 * DH), jnp.float32),
        mesh=_MESH,
        compiler_params=pltpu.CompilerParams(use_tc_tiling_on_sc=False),
        scratch_types=[
            pltpu.VMEM((nch, ch), jnp.int32),
            pltpu.VMEM((nch, ch), jnp.int32),
            pltpu.VMEM((nch, ch), jnp.int32),
            pltpu.VMEM((nch, ch), jnp.int32),
            pltpu.VMEM((ch, DH), jnp.float32),
            pltpu.VMEM((ch, DH), jnp.float32),
            pltpu.VMEM((ch, DH), jnp.float32),
            pltpu.VMEM((ch, DH), jnp.float32),
            pltpu.SemaphoreType.DMA((4,)),
            pltpu.SemaphoreType.DMA((4,)),
            pltpu.VMEM_SHARED((n, DH), jnp.float32),
        ],
    )
    return deg_k, agg_k


def _dis_col(degp_ref, nrows):
    deg = degp_ref[0] + degp_ref[1] + 1.0
    dis = lax.rsqrt(deg)
    return jnp.transpose(dis)[:nrows, :]


def _tc1_body(x_ref, w1_ref, degp_ref, h1s_ref):
    dis = _dis_col(degp_ref, x_ref.shape[0])
    h = jnp.dot(x_ref[...], w1_ref[...], preferred_element_type=jnp.float32)
    h1s_ref[...] = h * dis


def _tc2_body(agg_ref, h1s_ref, degp_ref, b1_ref, w2_ref, h2s_ref):
    dis = _dis_col(degp_ref, h1s_ref.shape[0])
    pre = (agg_ref[0] + agg_ref[1] + h1s_ref[...]) * dis + b1_ref[...]
    h1 = jnp.maximum(pre, 0.0)
    h2 = jnp.dot(h1, w2_ref[...], preferred_element_type=jnp.float32)
    h2s_ref[...] = h2 * dis


def _tc3_body(agg_ref, h2s_ref, degp_ref, b2_ref, out_ref):
    dis = _dis_col(degp_ref, h2s_ref.shape[0])
    out_ref[...] = (agg_ref[0] + agg_ref[1] + h2s_ref[...]) * dis \
        + b2_ref[...]


def kernel(x, edge_index, W1, b1, W2, b2):
    n, _ = x.shape
    d_hid = W1.shape[1]
    d_out = W2.shape[1]
    e = edge_index.shape[1]
    assert d_hid == 2 * DH and d_out == 2 * DH

    per_w = e // NW
    ch = 80
    while per_w % ch:
        ch -= 8
    nch = per_w // ch

    ei = edge_index.astype(jnp.int32)
    src = ei[0].reshape(NW, nch, ch)
    dst = ei[1].reshape(NW, nch, ch)
    n1 = (n + 127) // 128 * 128
    zeros1 = jnp.zeros((n1,), jnp.float32)
    zeros2 = jnp.zeros((n, DH), jnp.float32)

    deg_k, agg_k = _make_sc_kernels(n, nch, ch)

    degp = deg_k(dst, zeros1)
    degp2 = degp.reshape(NC, 1, n1)

    tc1 = pl.pallas_call(
        _tc1_body,
        out_shape=jax.ShapeDtypeStruct((n, d_hid), jnp.float32),
    )
    h1s = tc1(x, W1, degp2)

    agg1 = agg_k(h1s.reshape(2 * n, DH), src, dst, zeros2)

    tc2 = pl.pallas_call(
        _tc2_body,
        out_shape=jax.ShapeDtypeStruct((n, d_hid), jnp.float32),
    )
    h2s = tc2(agg1, h1s, degp2, b1.reshape(1, d_hid), W2)

    agg2 = agg_k(h2s.reshape(2 * n, DH), src, dst, zeros2)

    tc3 = pl.pallas_call(
        _tc3_body,
        out_shape=jax.ShapeDtypeStruct((n, d_out), jnp.float32),
    )
    out = tc3(agg2, h2s, degp2, b2.reshape(1, d_out))
    return out
